# baseline (speedup 1.0000x reference)
import jax
import jax.numpy as jnp
from jax import lax
from jax.experimental import pallas as pl
from jax.experimental.pallas import tpu as pltpu
from jax.experimental.pallas import tpu_sc as plsc

_B = 4096
_L = 50
_E = 32
_V = 100000

_NC = 2
_NS = 16
_NW = _NC * _NS
_RB = _B // _NW
_CB = 64
_NCHUNK = _RB // _CB
_G = 128
_NG = _CB * _L // _G

_LANES = 16


def _bow_body(x_ref, tab_ref, bow_ref, idx_v, rows_v, out_v, sem):
    wid = lax.axis_index("s") * _NC + lax.axis_index("c")
    inv_l = jnp.float32(1.0 / _L)

    pltpu.sync_copy(x_ref.at[wid], idx_v)

    for c in range(_NCHUNK):
        copies = []
        for g in range(_NG):
            copies.append(
                pltpu.async_copy(
                    tab_ref.at[idx_v.at[c * _NG + g]],
                    rows_v.at[pl.ds(g * _G, _G)],
                    sem,
                )
            )
        for cp in copies:
            cp.wait()

        def brow(i, carry):
            def jstep(j, acc):
                a0, a1 = acc
                r = i * _L + j
                a0 = a0 + rows_v[r, pl.ds(0, _LANES)]
                a1 = a1 + rows_v[r, pl.ds(_LANES, _LANES)]
                return (a0, a1)

            zero = jnp.zeros((_LANES,), jnp.float32)
            a0, a1 = lax.fori_loop(0, _L, jstep, (zero, zero))
            out_v[i, pl.ds(0, _LANES)] = a0 * inv_l
            out_v[i, pl.ds(_LANES, _LANES)] = a1 * inv_l
            return carry

        lax.fori_loop(0, _CB, brow, 0)

        pltpu.sync_copy(out_v, bow_ref.at[pl.ds(wid * _RB + c * _CB, _CB)])


def _bow_call(x2, emb_table):
    mesh = plsc.VectorSubcoreMesh(core_axis_name="c", subcore_axis_name="s")
    f = pl.kernel(
        _bow_body,
        out_type=jax.ShapeDtypeStruct((_B, _E), jnp.float32),
        mesh=mesh,
        scratch_types=[
            pltpu.VMEM((_RB * _L // _G, _G), jnp.int32),
            pltpu.VMEM((_CB * _L, _E), jnp.float32),
            pltpu.VMEM((_CB, _E), jnp.float32),
            pltpu.SemaphoreType.DMA,
        ],
        compiler_params=pltpu.CompilerParams(use_tc_tiling_on_sc=False),
    )
    return f(x2, emb_table)


_BM = 512
_BV = 4096


def _mm_body(bow_ref, w_ref, b_ref, out_ref):
    out_ref[...] = (
        lax.dot_general(
            bow_ref[...],
            w_ref[...],
            dimension_numbers=(((1,), (1,)), ((), ())),
            preferred_element_type=jnp.float32,
        )
        + b_ref[...]
    )


def _mm_call(bow, w, b2):
    return pl.pallas_call(
        _mm_body,
        grid=(_B // _BM, pl.cdiv(_V, _BV)),
        in_specs=[
            pl.BlockSpec((_BM, _E), lambda i, j: (i, 0)),
            pl.BlockSpec((_BV, _E), lambda i, j: (j, 0)),
            pl.BlockSpec((1, _BV), lambda i, j: (0, j)),
        ],
        out_specs=pl.BlockSpec((_BM, _BV), lambda i, j: (i, j)),
        out_shape=jax.ShapeDtypeStruct((_B, _V), jnp.float32),
        compiler_params=pltpu.CompilerParams(
            dimension_semantics=("parallel", "arbitrary"),
        ),
    )(bow, w, b2)


def kernel(X, emb_table, W, b):
    x2 = X.astype(jnp.int32).reshape(_NW, _RB * _L // _G, _G)
    bow = _bow_call(x2, emb_table)
    return _mm_call(bow, W, b.reshape(1, _V))


# speedup vs baseline: 1.0888x; 1.0888x over previous
import jax
import jax.numpy as jnp
from jax import lax
from jax.experimental import pallas as pl
from jax.experimental.pallas import tpu as pltpu
from jax.experimental.pallas import tpu_sc as plsc

_B = 4096
_L = 50
_E = 32
_V = 100000

_NC = 2
_NS = 16
_NW = _NC * _NS
_RB = _B // _NW
_CB = 64
_NCHUNK = _RB // _CB
_G = 128
_NG = _CB * _L // _G

_LANES = 16


def _bow_body(x_ref, tab_ref, bow_ref, idx_v, rows_v, out_v, sem):
    wid = lax.axis_index("s") * _NC + lax.axis_index("c")
    inv_l = jnp.float32(1.0 / _L)

    pltpu.sync_copy(x_ref.at[wid], idx_v)

    for c in range(_NCHUNK):
        copies = []
        for g in range(_NG):
            copies.append(
                pltpu.async_copy(
                    tab_ref.at[idx_v.at[c * _NG + g]],
                    rows_v.at[pl.ds(g * _G, _G)],
                    sem,
                )
            )
        for cp in copies:
            cp.wait()

        def brow(i, carry):
            def jstep(j, acc):
                a0, a1 = acc
                r = i * _L + j
                a0 = a0 + rows_v[r, pl.ds(0, _LANES)]
                a1 = a1 + rows_v[r, pl.ds(_LANES, _LANES)]
                return (a0, a1)

            zero = jnp.zeros((_LANES,), jnp.float32)
            a0, a1 = lax.fori_loop(0, _L, jstep, (zero, zero))
            out_v[i, pl.ds(0, _LANES)] = a0 * inv_l
            out_v[i, pl.ds(_LANES, _LANES)] = a1 * inv_l
            return carry

        lax.fori_loop(0, _CB, brow, 0)

        pltpu.sync_copy(out_v, bow_ref.at[pl.ds(wid * _RB + c * _CB, _CB)])


def _bow_call(x2, emb_table):
    mesh = plsc.VectorSubcoreMesh(core_axis_name="c", subcore_axis_name="s")
    f = pl.kernel(
        _bow_body,
        out_type=jax.ShapeDtypeStruct((_B, _E), jnp.float32),
        mesh=mesh,
        scratch_types=[
            pltpu.VMEM((_RB * _L // _G, _G), jnp.int32),
            pltpu.VMEM((_CB * _L, _E), jnp.float32),
            pltpu.VMEM((_CB, _E), jnp.float32),
            pltpu.SemaphoreType.DMA,
        ],
        compiler_params=pltpu.CompilerParams(use_tc_tiling_on_sc=False),
    )
    return f(x2, emb_table)


_BV = 1024


def _mm_body(bow_ref, wt_ref, b_ref, out_ref):
    out_ref[...] = (
        lax.dot_general(
            bow_ref[...],
            wt_ref[...],
            dimension_numbers=(((1,), (0,)), ((), ())),
            preferred_element_type=jnp.float32,
        )
        + b_ref[...]
    )


def _mm_call(bow, wt, b2):
    return pl.pallas_call(
        _mm_body,
        grid=(pl.cdiv(_V, _BV),),
        in_specs=[
            pl.BlockSpec((_B, _E), lambda j: (0, 0)),
            pl.BlockSpec((_E, _BV), lambda j: (0, j)),
            pl.BlockSpec((1, _BV), lambda j: (0, j)),
        ],
        out_specs=pl.BlockSpec((_B, _BV), lambda j: (0, j)),
        out_shape=jax.ShapeDtypeStruct((_B, _V), jnp.float32),
        compiler_params=pltpu.CompilerParams(
            dimension_semantics=("arbitrary",),
        ),
    )(bow, wt, b2)


def kernel(X, emb_table, W, b):
    x2 = X.astype(jnp.int32).reshape(_NW, _RB * _L // _G, _G)
    bow = _bow_call(x2, emb_table)
    return _mm_call(bow, W.T, b.reshape(1, _V))


# speedup vs baseline: 3.5830x; 3.2908x over previous
import jax
import jax.numpy as jnp
from jax import lax
from jax.experimental import pallas as pl
from jax.experimental.pallas import tpu as pltpu
from jax.experimental.pallas import tpu_sc as plsc

_B = 4096
_L = 50
_E = 32
_V = 100000

_NC = 2
_NS = 16
_NW = _NC * _NS
_RB = _B // _NW
_CB = 64
_NCHUNK = _RB // _CB
_G = 128
_NG = _CB * _L // _G

_LANES = 16


def _bow_body(x_ref, tab_ref, bow_ref, idx_v, rows_v, out_v, sem):
    wid = lax.axis_index("s") * _NC + lax.axis_index("c")
    inv_l = jnp.float32(1.0 / _L)

    pltpu.sync_copy(x_ref.at[wid], idx_v)

    for c in range(_NCHUNK):
        copies = []
        for g in range(_NG):
            copies.append(
                pltpu.async_copy(
                    tab_ref.at[idx_v.at[c * _NG + g]],
                    rows_v.at[pl.ds(g * _G, _G)],
                    sem,
                )
            )
        for cp in copies:
            cp.wait()

        def brow(i, carry):
            def jstep(j, acc):
                a0, a1 = acc
                r = i * _L + j
                a0 = a0 + rows_v[r, pl.ds(0, _LANES)]
                a1 = a1 + rows_v[r, pl.ds(_LANES, _LANES)]
                return (a0, a1)

            zero = jnp.zeros((_LANES,), jnp.float32)
            a0, a1 = lax.fori_loop(0, _L, jstep, (zero, zero))
            out_v[i, pl.ds(0, _LANES)] = a0 * inv_l
            out_v[i, pl.ds(_LANES, _LANES)] = a1 * inv_l
            return carry

        lax.fori_loop(0, _CB, brow, 0)

        pltpu.sync_copy(out_v, bow_ref.at[pl.ds(wid * _RB + c * _CB, _CB)])


def _bow_call(x2, emb_table):
    mesh = plsc.VectorSubcoreMesh(core_axis_name="c", subcore_axis_name="s")
    f = pl.kernel(
        _bow_body,
        out_type=jax.ShapeDtypeStruct((_B, _E), jnp.float32),
        mesh=mesh,
        scratch_types=[
            pltpu.VMEM((_RB * _L // _G, _G), jnp.int32),
            pltpu.VMEM((_CB * _L, _E), jnp.float32),
            pltpu.VMEM((_CB, _E), jnp.float32),
            pltpu.SemaphoreType.DMA,
        ],
        compiler_params=pltpu.CompilerParams(use_tc_tiling_on_sc=False),
    )
    return f(x2, emb_table)


_BV = 512
_K = _E + 1


def _mm_body(wb_ref, bow_ref, out_ref):
    out_ref[...] = lax.dot_general(
        wb_ref[...],
        bow_ref[...],
        dimension_numbers=(((0,), (1,)), ((), ())),
        preferred_element_type=jnp.float32,
    )


def _mm_call(wb, bow1):
    return pl.pallas_call(
        _mm_body,
        grid=(pl.cdiv(_V, _BV),),
        in_specs=[
            pl.BlockSpec((_K, _BV), lambda j: (0, j)),
            pl.BlockSpec((_B, _K), lambda j: (0, 0)),
        ],
        out_specs=pl.BlockSpec((_BV, _B), lambda j: (j, 0)),
        out_shape=jax.ShapeDtypeStruct((_V, _B), jnp.float32),
        compiler_params=pltpu.CompilerParams(
            dimension_semantics=("arbitrary",),
        ),
    )(wb, bow1)


def kernel(X, emb_table, W, b):
    x2 = X.astype(jnp.int32).reshape(_NW, _RB * _L // _G, _G)
    bow = _bow_call(x2, emb_table)
    wb = jnp.concatenate([W.T, b[None, :]], axis=0)
    bow1 = jnp.concatenate([bow, jnp.ones((_B, 1), jnp.float32)], axis=1)
    return _mm_call(wb, bow1).T


# speedup vs baseline: 3.5970x; 1.0039x over previous
import jax
import jax.numpy as jnp
from jax import lax
from jax.experimental import pallas as pl
from jax.experimental.pallas import tpu as pltpu
from jax.experimental.pallas import tpu_sc as plsc

_B = 4096
_L = 50
_E = 32
_V = 100000

_NC = 2
_NS = 16
_NW = _NC * _NS
_RB = _B // _NW
_CB = 64
_NCHUNK = _RB // _CB
_G = 128
_NG = _CB * _L // _G

_LANES = 16


def _bow_body(x_ref, tab_ref, bow_ref, idx_v, rows_v, out_v, sem):
    wid = lax.axis_index("s") * _NC + lax.axis_index("c")
    inv_l = jnp.float32(1.0 / _L)

    pltpu.sync_copy(x_ref.at[wid], idx_v)

    for c in range(_NCHUNK):
        copies = []
        for g in range(_NG):
            copies.append(
                pltpu.async_copy(
                    tab_ref.at[idx_v.at[c * _NG + g]],
                    rows_v.at[pl.ds(g * _G, _G)],
                    sem,
                )
            )
        for cp in copies:
            cp.wait()

        def brow(i, carry):
            def jstep(j, acc):
                a0, a1 = acc
                r = i * _L + j
                a0 = a0 + rows_v[r, pl.ds(0, _LANES)]
                a1 = a1 + rows_v[r, pl.ds(_LANES, _LANES)]
                return (a0, a1)

            zero = jnp.zeros((_LANES,), jnp.float32)
            a0, a1 = lax.fori_loop(0, _L, jstep, (zero, zero))
            out_v[i, pl.ds(0, _LANES)] = a0 * inv_l
            out_v[i, pl.ds(_LANES, _LANES)] = a1 * inv_l
            return carry

        lax.fori_loop(0, _CB, brow, 0)

        pltpu.sync_copy(out_v, bow_ref.at[pl.ds(wid * _RB + c * _CB, _CB)])


def _bow_call(x2, emb_table):
    mesh = plsc.VectorSubcoreMesh(core_axis_name="c", subcore_axis_name="s")
    f = pl.kernel(
        _bow_body,
        out_type=jax.ShapeDtypeStruct((_B, _E), jnp.float32),
        mesh=mesh,
        scratch_types=[
            pltpu.VMEM((_RB * _L // _G, _G), jnp.int32),
            pltpu.VMEM((_CB * _L, _E), jnp.float32),
            pltpu.VMEM((_CB, _E), jnp.float32),
            pltpu.SemaphoreType.DMA,
        ],
        compiler_params=pltpu.CompilerParams(use_tc_tiling_on_sc=False),
    )
    return f(x2, emb_table)


_BV = 1024
_K = _E + 1


def _mm_body(wb_ref, bow_ref, out_ref):
    out_ref[...] = lax.dot_general(
        wb_ref[...],
        bow_ref[...],
        dimension_numbers=(((0,), (1,)), ((), ())),
        preferred_element_type=jnp.float32,
    )


def _mm_call(wb, bow1):
    return pl.pallas_call(
        _mm_body,
        grid=(pl.cdiv(_V, _BV),),
        in_specs=[
            pl.BlockSpec((_K, _BV), lambda j: (0, j)),
            pl.BlockSpec((_B, _K), lambda j: (0, 0)),
        ],
        out_specs=pl.BlockSpec((_BV, _B), lambda j: (j, 0)),
        out_shape=jax.ShapeDtypeStruct((_V, _B), jnp.float32),
        compiler_params=pltpu.CompilerParams(
            dimension_semantics=("arbitrary",),
        ),
    )(wb, bow1)


def kernel(X, emb_table, W, b):
    x2 = X.astype(jnp.int32).reshape(_NW, _RB * _L // _G, _G)
    bow = _bow_call(x2, emb_table)
    wb = jnp.concatenate([W.T, b[None, :]], axis=0)
    bow1 = jnp.concatenate([bow, jnp.ones((_B, 1), jnp.float32)], axis=1)
    return _mm_call(wb, bow1).T


# speedup vs baseline: 3.6426x; 1.0127x over previous
import jax
import jax.numpy as jnp
from jax import lax
from jax.experimental import pallas as pl
from jax.experimental.pallas import tpu as pltpu
from jax.experimental.pallas import tpu_sc as plsc

_B = 4096
_L = 50
_E = 32
_V = 100000

_NC = 2
_NS = 16
_NW = _NC * _NS
_RB = _B // _NW
_CB = 32
_NCHUNK = _RB // _CB
_G = 128

_LANES = 16


_CI = _CB * _L
_GATHERS = [_G] * (_CI // _G) + ([_CI % _G] if _CI % _G else [])


def _bow_body(x_ref, tab_ref, bow_ref, idx_v, rows_v, out_v, sem0, sem1):
    wid = lax.axis_index("s") * _NC + lax.axis_index("c")
    inv_l = jnp.float32(1.0 / _L)
    sems = (sem0, sem1)

    pltpu.sync_copy(x_ref.at[pl.ds(wid * _RB * _L, _RB * _L)], idx_v)

    def fire(c):
        buf = c % 2
        copies = []
        off = 0
        for g in _GATHERS:
            copies.append(
                pltpu.async_copy(
                    tab_ref.at[idx_v.at[pl.ds(c * _CI + off, g)]],
                    rows_v.at[buf, pl.ds(off, g)],
                    sems[buf],
                )
            )
            off += g
        return copies

    inflight = fire(0)
    for c in range(_NCHUNK):
        for cp in inflight:
            cp.wait()
        if c + 1 < _NCHUNK:
            inflight = fire(c + 1)
        buf = c % 2

        def brow(i, carry):
            def jstep(j, acc):
                a0, a1 = acc
                r = i * _L + j
                a0 = a0 + rows_v[buf, r, pl.ds(0, _LANES)]
                a1 = a1 + rows_v[buf, r, pl.ds(_LANES, _LANES)]
                return (a0, a1)

            zero = jnp.zeros((_LANES,), jnp.float32)
            a0, a1 = lax.fori_loop(0, _L, jstep, (zero, zero))
            out_v[c * _CB + i, pl.ds(0, _LANES)] = a0 * inv_l
            out_v[c * _CB + i, pl.ds(_LANES, _LANES)] = a1 * inv_l
            return carry

        lax.fori_loop(0, _CB, brow, 0)

    pltpu.sync_copy(out_v, bow_ref.at[pl.ds(wid * _RB, _RB)])


def _bow_call(x1, emb_table):
    mesh = plsc.VectorSubcoreMesh(core_axis_name="c", subcore_axis_name="s")
    f = pl.kernel(
        _bow_body,
        out_type=jax.ShapeDtypeStruct((_B, _E), jnp.float32),
        mesh=mesh,
        scratch_types=[
            pltpu.VMEM((_RB * _L,), jnp.int32),
            pltpu.VMEM((2, _CI, _E), jnp.float32),
            pltpu.VMEM((_RB, _E), jnp.float32),
            pltpu.SemaphoreType.DMA,
            pltpu.SemaphoreType.DMA,
        ],
        compiler_params=pltpu.CompilerParams(use_tc_tiling_on_sc=False),
    )
    return f(x1, emb_table)


_BV = 1024
_K = _E + 1


def _mm_body(wb_ref, bow_ref, out_ref):
    out_ref[...] = lax.dot_general(
        wb_ref[...],
        bow_ref[...],
        dimension_numbers=(((0,), (1,)), ((), ())),
        preferred_element_type=jnp.float32,
    )


def _mm_call(wb, bow1):
    return pl.pallas_call(
        _mm_body,
        grid=(pl.cdiv(_V, _BV),),
        in_specs=[
            pl.BlockSpec((_K, _BV), lambda j: (0, j)),
            pl.BlockSpec((_B, _K), lambda j: (0, 0)),
        ],
        out_specs=pl.BlockSpec((_BV, _B), lambda j: (j, 0)),
        out_shape=jax.ShapeDtypeStruct((_V, _B), jnp.float32),
        compiler_params=pltpu.CompilerParams(
            dimension_semantics=("arbitrary",),
        ),
    )(wb, bow1)


def kernel(X, emb_table, W, b):
    x1 = X.astype(jnp.int32).reshape(_B * _L)
    bow = _bow_call(x1, emb_table)
    wb = jnp.concatenate([W.T, b[None, :]], axis=0)
    bow1 = jnp.concatenate([bow, jnp.ones((_B, 1), jnp.float32)], axis=1)
    return _mm_call(wb, bow1).T
